# baseline (speedup 1.0000x reference)
import jax
import jax.numpy as jnp
from jax import lax
from jax.experimental import pallas as pl
from jax.experimental.pallas import tpu as pltpu

NBINS = 15


def _ece_block(x_ref, t_ref, o_ref, cnt_ref, acc_ref, cf_ref):
    i = pl.program_id(0)
    nb = pl.num_programs(0)

    @pl.when(i == 0)
    def _init():
        cnt_ref[:] = jnp.zeros_like(cnt_ref)
        acc_ref[:] = jnp.zeros_like(acc_ref)
        cf_ref[:] = jnp.zeros_like(cf_ref)

    x = x_ref[:]
    R, C = x.shape
    conf = jnp.max(x, axis=1)
    col = lax.broadcasted_iota(jnp.int32, (R, C), 1)
    pred = jnp.min(jnp.where(x == conf[:, None], col, C), axis=1)
    tgt = t_ref[0, 0, :]
    correct = (pred == tgt).astype(jnp.float32)
    binid = jnp.clip(jnp.floor(conf * NBINS).astype(jnp.int32), 0, NBINS - 1)

    L = R // 128
    conf2 = conf.reshape(L, 128)
    cor2 = correct.reshape(L, 128)
    bin2 = binid.reshape(L, 128)
    for b in range(NBINS):
        m = bin2 == b
        cnt_ref[b, :] += jnp.sum(jnp.where(m, 1.0, 0.0), axis=0)
        acc_ref[b, :] += jnp.sum(jnp.where(m, cor2, 0.0), axis=0)
        cf_ref[b, :] += jnp.sum(jnp.where(m, conf2, 0.0), axis=0)

    @pl.when(i == nb - 1)
    def _fin():
        n_total = nb * R
        counts = jnp.sum(cnt_ref[:], axis=1)
        accs = jnp.sum(acc_ref[:], axis=1)
        confs = jnp.sum(cf_ref[:], axis=1)
        safe = jnp.maximum(counts, 1.0)
        per_bin = jnp.where(
            counts > 0,
            (counts / n_total) * jnp.abs(accs / safe - confs / safe),
            0.0,
        )
        o_ref[:, :] = jnp.full((1, 128), jnp.sum(per_bin), jnp.float32)


def kernel(inputs, targets):
    N, C = inputs.shape
    R = min(8192, N)
    NB = N // R
    tgt3 = targets.astype(jnp.int32).reshape(NB, 1, R)
    out = pl.pallas_call(
        _ece_block,
        grid=(NB,),
        in_specs=[
            pl.BlockSpec((R, C), lambda i: (i, 0)),
            pl.BlockSpec((1, 1, R), lambda i: (i, 0, 0)),
        ],
        out_specs=pl.BlockSpec((1, 128), lambda i: (0, 0)),
        out_shape=jax.ShapeDtypeStruct((1, 128), jnp.float32),
        scratch_shapes=[
            pltpu.VMEM((16, 128), jnp.float32),
            pltpu.VMEM((16, 128), jnp.float32),
            pltpu.VMEM((16, 128), jnp.float32),
        ],
    )(inputs, tgt3)
    return out[0, 0].reshape(())


# speedup vs baseline: 7.9920x; 7.9920x over previous
import jax
import jax.numpy as jnp
from jax import lax
from jax.experimental import pallas as pl
from jax.experimental.pallas import tpu as pltpu

NBINS = 15


def _ece_block(x_ref, t_ref, o_ref, cnt_ref, acc_ref, cf_ref):
    i = pl.program_id(0)
    nb = pl.num_programs(0)

    @pl.when(i == 0)
    def _init():
        cnt_ref[:] = jnp.zeros_like(cnt_ref)
        acc_ref[:] = jnp.zeros_like(acc_ref)
        cf_ref[:] = jnp.zeros_like(cf_ref)

    x = x_ref[:]
    R, C = x.shape
    T = R // 128
    x3 = x.reshape(T, 128, C)
    xt = jnp.transpose(x3, (0, 2, 1))
    conf_k = jnp.max(xt, axis=1, keepdims=True)
    cls = lax.broadcasted_iota(jnp.int32, (T, C, 128), 1)
    pred = jnp.min(jnp.where(xt == conf_k, cls, C), axis=1)
    conf = conf_k[:, 0, :]
    tgt = t_ref[0, 0, :].reshape(T, 128)
    correct = (pred == tgt).astype(jnp.float32)
    binid = jnp.clip(jnp.floor(conf * NBINS).astype(jnp.int32), 0, NBINS - 1)

    ones = jnp.ones_like(conf)
    zero = jnp.zeros_like(conf)
    for b in range(NBINS):
        m = binid == b
        s = slice(b * T, (b + 1) * T)
        cnt_ref[s, :] += jnp.where(m, ones, zero)
        acc_ref[s, :] += jnp.where(m, correct, zero)
        cf_ref[s, :] += jnp.where(m, conf, zero)

    @pl.when(i == nb - 1)
    def _fin():
        n_total = nb * R
        counts = jnp.sum(cnt_ref[:].reshape(NBINS, T, 128), axis=(1, 2))
        accs = jnp.sum(acc_ref[:].reshape(NBINS, T, 128), axis=(1, 2))
        confs = jnp.sum(cf_ref[:].reshape(NBINS, T, 128), axis=(1, 2))
        safe = jnp.maximum(counts, 1.0)
        per_bin = jnp.where(
            counts > 0,
            (counts / n_total) * jnp.abs(accs / safe - confs / safe),
            0.0,
        )
        o_ref[:, :] = jnp.full((1, 128), jnp.sum(per_bin), jnp.float32)


def kernel(inputs, targets):
    N, C = inputs.shape
    R = min(2048, N)
    NB = N // R
    T = R // 128
    tgt3 = targets.astype(jnp.int32).reshape(NB, 1, R)
    out = pl.pallas_call(
        _ece_block,
        grid=(NB,),
        in_specs=[
            pl.BlockSpec((R, C), lambda i: (i, 0)),
            pl.BlockSpec((1, 1, R), lambda i: (i, 0, 0)),
        ],
        out_specs=pl.BlockSpec((1, 128), lambda i: (0, 0)),
        out_shape=jax.ShapeDtypeStruct((1, 128), jnp.float32),
        scratch_shapes=[
            pltpu.VMEM((NBINS * T, 128), jnp.float32),
            pltpu.VMEM((NBINS * T, 128), jnp.float32),
            pltpu.VMEM((NBINS * T, 128), jnp.float32),
        ],
    )(inputs, tgt3)
    return out[0, 0].reshape(())


# speedup vs baseline: 12.9260x; 1.6174x over previous
import jax
import jax.numpy as jnp
from jax import lax
from jax.experimental import pallas as pl
from jax.experimental.pallas import tpu as pltpu

NBINS = 15


def _rowmax_argmax(xt, ci):
    v3 = xt.reshape(16, 8, 128)
    v = v3
    while v.shape[0] > 1:
        h = v.shape[0] // 2
        v = jnp.maximum(v[:h], v[h:])
    v = v[0]
    for k in (4, 2, 1):
        v = jnp.maximum(v, jnp.roll(v, k, axis=0))
    m = v3 == v[None, :, :]
    w = jnp.where(m, ci, 3.0e4)
    while w.shape[0] > 1:
        h = w.shape[0] // 2
        w = jnp.minimum(w[:h], w[h:])
    w = w[0]
    for k in (4, 2, 1):
        w = jnp.minimum(w, jnp.roll(w, k, axis=0))
    return v[0:1, :], w[0:1, :]


def _ece_block(x_ref, t_ref, o_ref, cnt_ref, acc_ref, cf_ref):
    i = pl.program_id(0)
    nb = pl.num_programs(0)

    @pl.when(i == 0)
    def _init():
        cnt_ref[:] = jnp.zeros_like(cnt_ref)
        acc_ref[:] = jnp.zeros_like(acc_ref)
        cf_ref[:] = jnp.zeros_like(cf_ref)

    x = x_ref[:]
    R, C = x.shape
    T = R // 128
    ci = (lax.broadcasted_iota(jnp.int32, (16, 8, 128), 0) * 8
          + lax.broadcasted_iota(jnp.int32, (16, 8, 128), 1)).astype(jnp.float32)
    confs = []
    preds = []
    for t in range(T):
        xt = x[t * 128:(t + 1) * 128, :].T
        c_t, p_t = _rowmax_argmax(xt, ci)
        confs.append(c_t)
        preds.append(p_t)
    conf = jnp.concatenate(confs, axis=0)
    pred = jnp.concatenate(preds, axis=0)
    tgt = t_ref[0, 0, :].reshape(T, 128).astype(jnp.float32)
    correct = (pred == tgt).astype(jnp.float32)
    binid = jnp.clip(jnp.floor(conf * NBINS).astype(jnp.int32), 0, NBINS - 1)

    ones = jnp.ones_like(conf)
    zero = jnp.zeros_like(conf)
    for b in range(NBINS):
        m = binid == b
        s = slice(b * T, (b + 1) * T)
        cnt_ref[s, :] += jnp.where(m, ones, zero)
        acc_ref[s, :] += jnp.where(m, correct, zero)
        cf_ref[s, :] += jnp.where(m, conf, zero)

    @pl.when(i == nb - 1)
    def _fin():
        n_total = nb * R
        counts = jnp.sum(cnt_ref[:].reshape(NBINS, T, 128), axis=(1, 2))
        accs = jnp.sum(acc_ref[:].reshape(NBINS, T, 128), axis=(1, 2))
        confs_ = jnp.sum(cf_ref[:].reshape(NBINS, T, 128), axis=(1, 2))
        safe = jnp.maximum(counts, 1.0)
        per_bin = jnp.where(
            counts > 0,
            (counts / n_total) * jnp.abs(accs / safe - confs_ / safe),
            0.0,
        )
        o_ref[:, :] = jnp.full((1, 128), jnp.sum(per_bin), jnp.float32)


def kernel(inputs, targets):
    N, C = inputs.shape
    R = min(4096, N)
    NB = N // R
    T = R // 128
    tgt3 = targets.astype(jnp.int32).reshape(NB, 1, R)
    out = pl.pallas_call(
        _ece_block,
        grid=(NB,),
        in_specs=[
            pl.BlockSpec((R, C), lambda i: (i, 0)),
            pl.BlockSpec((1, 1, R), lambda i: (i, 0, 0)),
        ],
        out_specs=pl.BlockSpec((1, 128), lambda i: (0, 0)),
        out_shape=jax.ShapeDtypeStruct((1, 128), jnp.float32),
        scratch_shapes=[
            pltpu.VMEM((NBINS * T, 128), jnp.float32),
            pltpu.VMEM((NBINS * T, 128), jnp.float32),
            pltpu.VMEM((NBINS * T, 128), jnp.float32),
        ],
    )(inputs, tgt3)
    return out[0, 0].reshape(())
